# baseline (speedup 1.0000x reference)
import functools

import jax
import jax.numpy as jnp
from jax.experimental import pallas as pl
from jax.experimental.pallas import tpu as pltpu


def _linear_body(a_ref, w_ref, b_ref, o_ref, *, relu):
    acc = jnp.dot(a_ref[...], w_ref[...], preferred_element_type=jnp.float32)
    acc = acc + b_ref[...]
    if relu:
        acc = jnp.maximum(acc, 0.0)
    o_ref[...] = acc


@functools.partial(jax.jit, static_argnames=("relu",))
def _linear(a, w, b, relu=False):
    m, k = a.shape
    _, n = w.shape
    bm = 2000
    grid = (m // bm,)
    return pl.pallas_call(
        functools.partial(_linear_body, relu=relu),
        grid=grid,
        in_specs=[
            pl.BlockSpec((bm, k), lambda i: (i, 0)),
            pl.BlockSpec((k, n), lambda i: (0, 0)),
            pl.BlockSpec((n,), lambda i: (0,)),
        ],
        out_specs=pl.BlockSpec((bm, n), lambda i: (i, 0)),
        out_shape=jax.ShapeDtypeStruct((m, n), jnp.float32),
    )(a, w, b)


def kernel(x, edge_index, params):
    n = x.shape[0]
    loop = jnp.arange(n, dtype=edge_index.dtype)
    src = jnp.concatenate([edge_index[0], loop])
    dst = jnp.concatenate([edge_index[1], loop])
    deg = jax.ops.segment_sum(jnp.ones_like(dst, dtype=jnp.float32), dst,
                              num_segments=n)
    dinv = jax.lax.rsqrt(jnp.maximum(deg, 1.0))
    norm = dinv[src] * dinv[dst]

    def gcn(h, p):
        hw = _linear(h, p["W"], jnp.zeros_like(p["b"]))
        msg = hw[src] * norm[:, None]
        return jax.ops.segment_sum(msg, dst, num_segments=n) + p["b"]

    def dbl(h, p1, p2):
        return jax.nn.relu(gcn(jax.nn.relu(gcn(h, p1)), p2))

    enc_outs = []
    h = dbl(x, params["enc0_c1"], params["enc0_c2"])
    enc_outs.append(h)
    h = _linear(h, params["pool0"]["W"], params["pool0"]["b"], relu=True)
    h = dbl(h, params["enc1_c1"], params["enc1_c2"])
    enc_outs.append(h)
    h = _linear(h, params["pool1"]["W"], params["pool1"]["b"], relu=True)
    h = dbl(h, params["bot_c1"], params["bot_c2"])
    h = _linear(h, params["up0"]["W"], params["up0"]["b"], relu=True)
    h = jnp.concatenate([h, enc_outs.pop()], axis=1)
    h = dbl(h, params["dec0_c1"], params["dec0_c2"])
    out = _linear(h, params["out"]["W"], params["out"]["b"])
    return jax.nn.sigmoid(out)


# speedup vs baseline: 3.1632x; 3.1632x over previous
import functools

import jax
import jax.numpy as jnp
from jax import lax
from jax.experimental import pallas as pl
from jax.experimental.pallas import tpu as pltpu
from jax.experimental.pallas import tpu_sc as plsc

N = 10000
NP = 10240
E = 160000
_TILES = 16
_NC = 2
_B = 128
_CH = 80
_HC = _CH // 2
_RPT = NP // _TILES
_FC = 128

_mesh = plsc.VectorSubcoreMesh(core_axis_name="c", subcore_axis_name="s")


def _fill(buf, rows, cols, val):
    v = jnp.full((16,), val, jnp.float32)

    def row(i, _):
        for j in range(cols // 16):
            buf[i, pl.ds(j * 16, 16)] = v
        return 0

    lax.fori_loop(0, rows, row, 0)


def _make_prop(S):
    scratch = [
        pltpu.VMEM_SHARED((NP, _FC), jnp.float32),
        pltpu.VMEM((_HC, _B), jnp.int32),
        pltpu.VMEM((_HC, _B), jnp.int32),
        pltpu.VMEM((_B, _FC), jnp.float32),
        pltpu.SemaphoreType.DMA,
    ]
    out_type = [jax.ShapeDtypeStruct((_NC, NP, _FC), jnp.float32)] * S

    @functools.partial(pl.kernel, out_type=out_type, mesh=_mesh,
                       scratch_types=scratch, name=f"sc_prop{S}")
    def prop(src_h, dst_h, *rest):
        gs = rest[0:S]
        outs = rest[S:2 * S]
        acc, isrc, idst, rows, sem = rest[2 * S:]
        c = lax.axis_index("c")
        t = lax.axis_index("s")
        row0 = t * _RPT
        pltpu.sync_copy(src_h.at[t, pl.ds(c * _HC, _HC)], isrc)
        pltpu.sync_copy(dst_h.at[t, pl.ds(c * _HC, _HC)], idst)

        for si in range(S):
            g = gs[si]
            o = outs[si]

            _fill(rows, _B, _FC, 0.0)
            for z in range(_RPT // _B):
                pltpu.sync_copy(rows, acc.at[pl.ds(row0 + z * _B, _B)])
            plsc.subcore_barrier()

            def chunk(j, _, g=g):
                pltpu.async_copy(g.at[isrc.at[j]], rows, sem).wait()
                pltpu.sync_copy(rows, acc.at[idst.at[j]], add=True)
                return 0

            lax.fori_loop(0, _HC, chunk, 0)
            plsc.subcore_barrier()
            for z in range(_RPT // _B):
                pltpu.sync_copy(acc.at[pl.ds(row0 + z * _B, _B)], rows)
                pltpu.sync_copy(rows, o.at[c, pl.ds(row0 + z * _B, _B)])

    return prop


_prop2 = _make_prop(2)


def _prop(src3, dst3, gs):
    outs = []
    for i in range(0, len(gs), 2):
        outs += _prop2(src3, dst3, gs[i], gs[i + 1])
    return outs


@functools.partial(
    pl.kernel,
    out_type=jax.ShapeDtypeStruct((_NC, NP, _FC), jnp.float32),
    mesh=_mesh,
    scratch_types=[
        pltpu.VMEM_SHARED((NP, _FC), jnp.float32),
        pltpu.VMEM((_HC, _B), jnp.int32),
        pltpu.VMEM((_B, _FC), jnp.float32),
    ],
    name="sc_deg")
def _degk(dst_h, out, acc, idst, onesb):
    c = lax.axis_index("c")
    t = lax.axis_index("s")
    row0 = t * _RPT
    pltpu.sync_copy(dst_h.at[t, pl.ds(c * _HC, _HC)], idst)
    _fill(onesb, _B, _FC, 0.0)
    for z in range(_RPT // _B):
        pltpu.sync_copy(onesb, acc.at[pl.ds(row0 + z * _B, _B)])
    _fill(onesb, _B, _FC, 1.0)
    plsc.subcore_barrier()

    def chunk(j, _):
        pltpu.sync_copy(onesb, acc.at[idst.at[j]], add=True)
        return 0

    lax.fori_loop(0, _HC, chunk, 0)
    plsc.subcore_barrier()
    for z in range(_RPT // _B):
        pltpu.sync_copy(acc.at[pl.ds(row0 + z * _B, _B)], onesb)
        pltpu.sync_copy(onesb, out.at[c, pl.ds(row0 + z * _B, _B)])


_BM = 2048


def _tc(body, n_out_cols, arrays):
    grid = (NP // _BM,)

    def spec(a):
        if a.ndim == 2 and a.shape[0] == NP:
            return pl.BlockSpec((_BM, a.shape[1]), lambda i: (i, 0))
        return pl.BlockSpec(a.shape, lambda i: (0,) * a.ndim)

    out_shape = [jax.ShapeDtypeStruct((NP, ncol), jnp.float32)
                 for ncol in n_out_cols]
    out_specs = [pl.BlockSpec((_BM, ncol), lambda i: (i, 0))
                 for ncol in n_out_cols]
    res = pl.pallas_call(
        body,
        grid=grid,
        in_specs=[spec(a) for a in arrays],
        out_specs=out_specs,
        out_shape=out_shape,
    )(*arrays)
    return res


def _dot(a, b):
    return jnp.dot(a, b, preferred_element_type=jnp.float32)


def _cat_combine(dv, b, sa, sb, gvals):
    parts = [dv * (x + y + g) for x, y, g in zip(sa, sb, gvals)]
    return jnp.maximum(jnp.concatenate(parts, axis=1) + b, 0.0)


def _split_out(g, outs):
    for i, o in enumerate(outs):
        o[...] = g[:, _FC * i:_FC * (i + 1)]


def _dinv_body(d0_ref, d1_ref, o_ref):
    deg = d0_ref[...][:, :1] + d1_ref[...][:, :1]
    o_ref[...] = lax.rsqrt(jnp.maximum(deg + 1.0, 1.0))


def _mm_scale_body(S):
    def body(h_ref, w_ref, dv_ref, *outs):
        g = _dot(h_ref[...], w_ref[...]) * dv_ref[...]
        _split_out(g, outs)
    return body


def _unpack_comb(refs, S_in):
    dv = refs[0][...]
    b = refs[1][...]
    sa = [r[...] for r in refs[2:2 + S_in]]
    sb = [r[...] for r in refs[2 + S_in:2 + 2 * S_in]]
    gvals = [r[...] for r in refs[2 + 2 * S_in:2 + 3 * S_in]]
    return dv, b, sa, sb, gvals, 2 + 3 * S_in


def _comb_mm_scale_body(S_in):
    def body(*refs):
        dv, b, sa, sb, gvals, k = _unpack_comb(refs, S_in)
        w = refs[k][...]
        outs = refs[k + 1:]
        h = _cat_combine(dv, b, sa, sb, gvals)
        _split_out(_dot(h, w) * dv, outs)
    return body


def _comb_pool_mm_scale_body(S_in, want_h):
    def body(*refs):
        dv, b, sa, sb, gvals, k = _unpack_comb(refs, S_in)
        wp, bp, wn = (r[...] for r in refs[k:k + 3])
        k += 3
        h = _cat_combine(dv, b, sa, sb, gvals)
        if want_h:
            refs[k][...] = h
            k += 1
        outs = refs[k:]
        hp = jnp.maximum(_dot(h, wp) + bp, 0.0)
        _split_out(_dot(hp, wn) * dv, outs)
    return body


def _comb_up_cat_mm_scale_body(S_in):
    def body(*refs):
        dv, b, sa, sb, gvals, k = _unpack_comb(refs, S_in)
        wu, bu, wa, wb, e1 = (r[...] for r in refs[k:k + 5])
        outs = refs[k + 5:]
        h = _cat_combine(dv, b, sa, sb, gvals)
        hu = jnp.maximum(_dot(h, wu) + bu, 0.0)
        g = (_dot(hu, wa) + _dot(e1, wb)) * dv
        _split_out(g, outs)
    return body


def _comb_out_body(S_in):
    def body(*refs):
        dv, b, sa, sb, gvals, k = _unpack_comb(refs, S_in)
        wo, bo = (r[...] for r in refs[k:k + 2])
        out = refs[k + 2]
        h = _cat_combine(dv, b, sa, sb, gvals)
        out[...] = jax.nn.sigmoid(_dot(h, wo) + bo)
    return body


def kernel(x, edge_index, params):
    p = params

    xp = jnp.zeros((NP, x.shape[1]), jnp.float32).at[:N].set(x)
    pad = _TILES * _CH * _B - E
    src3 = jnp.concatenate(
        [edge_index[0], jnp.full((pad,), NP - 1, edge_index.dtype)]
    ).reshape(_TILES, _CH, _B)
    dst3 = jnp.concatenate(
        [edge_index[1], jnp.full((pad,), NP - 1, edge_index.dtype)]
    ).reshape(_TILES, _CH, _B)

    def b2(name):
        return p[name]["b"].reshape(1, -1)

    def ab(s):
        return [u[0] for u in s], [u[1] for u in s]

    d2 = _degk(dst3)
    dv = _tc(_dinv_body, [1], [d2[0], d2[1]])[0]

    g1 = _tc(_mm_scale_body(4), [_FC] * 4, [xp, p["enc0_c1"]["W"], dv])
    a1, b1 = ab(_prop(src3, dst3, g1))
    g2 = _tc(_comb_mm_scale_body(4), [_FC] * 4,
             [dv, b2("enc0_c1"), *a1, *b1, *g1, p["enc0_c2"]["W"]])
    a2, b2_ = ab(_prop(src3, dst3, g2))
    g3 = _tc(_comb_pool_mm_scale_body(4, False), [_FC] * 4,
             [dv, b2("enc0_c2"), *a2, *b2_, *g2,
              p["pool0"]["W"], b2("pool0"), p["enc1_c1"]["W"]])

    a3, b3 = ab(_prop(src3, dst3, g3))
    g4 = _tc(_comb_mm_scale_body(4), [_FC] * 4,
             [dv, b2("enc1_c1"), *a3, *b3, *g3, p["enc1_c2"]["W"]])
    a4, b4 = ab(_prop(src3, dst3, g4))
    r5 = _tc(_comb_pool_mm_scale_body(4, True), [512] + [_FC] * 4,
             [dv, b2("enc1_c2"), *a4, *b4, *g4,
              p["pool1"]["W"], b2("pool1"), p["bot_c1"]["W"]])
    e1, g5 = r5[0], r5[1:]

    a5, b5 = ab(_prop(src3, dst3, list(g5)))
    g6 = _tc(_comb_mm_scale_body(4), [_FC] * 4,
             [dv, b2("bot_c1"), *a5, *b5, *g5, p["bot_c2"]["W"]])
    a6, b6 = ab(_prop(src3, dst3, g6))

    wd = p["dec0_c1"]["W"]
    wa, wb = wd[:256], wd[256:]
    g7 = _tc(_comb_up_cat_mm_scale_body(4), [_FC] * 2,
             [dv, b2("bot_c2"), *a6, *b6, *g6,
              p["up0"]["W"], b2("up0"), wa, wb, e1])
    a7, b7 = ab(_prop(src3, dst3, g7))
    g8 = _tc(_comb_mm_scale_body(2), [_FC] * 2,
             [dv, b2("dec0_c1"), *a7, *b7, *g7, p["dec0_c2"]["W"]])
    a8, b8 = ab(_prop(src3, dst3, g8))
    out = _tc(_comb_out_body(2), [256],
              [dv, b2("dec0_c2"), *a8, *b8, *g8, p["out"]["W"], b2("out")])[0]
    return out[:N]


# speedup vs baseline: 3.5818x; 1.1324x over previous
import functools

import jax
import jax.numpy as jnp
from jax import lax
from jax.experimental import pallas as pl
from jax.experimental.pallas import tpu as pltpu
from jax.experimental.pallas import tpu_sc as plsc

N = 10000
NP = 10240
E = 160000
_TILES = 16
_NC = 2
_B = 128
_CH = 80
_HC = _CH // 2
_RPT = NP // _TILES
_FC = 128

_mesh = plsc.VectorSubcoreMesh(core_axis_name="c", subcore_axis_name="s")


def _fill(buf, rows, cols, val):
    v = jnp.full((16,), val, jnp.float32)

    def row(i, _):
        for j in range(cols // 16):
            buf[i, pl.ds(j * 16, 16)] = v
        return 0

    lax.fori_loop(0, rows, row, 0)


def _make_prop(S):
    scratch = [
        pltpu.VMEM_SHARED((NP, _FC), jnp.float32),
        pltpu.VMEM((_HC, _B), jnp.int32),
        pltpu.VMEM((_HC, _B), jnp.int32),
        pltpu.VMEM((_B, _FC), jnp.float32),
        pltpu.VMEM((_B, _FC), jnp.float32),
        pltpu.SemaphoreType.DMA,
        pltpu.SemaphoreType.DMA,
    ]
    out_type = [jax.ShapeDtypeStruct((_NC, NP, _FC), jnp.float32)] * S
    _HP = _HC // 2

    @functools.partial(pl.kernel, out_type=out_type, mesh=_mesh,
                       scratch_types=scratch, name=f"sc_prop{S}")
    def prop(src_h, dst_h, *rest):
        gs = rest[0:S]
        outs = rest[S:2 * S]
        acc, isrc, idst, buf0, buf1, sem0, sem1 = rest[2 * S:]
        c = lax.axis_index("c")
        t = lax.axis_index("s")
        row0 = t * _RPT
        pltpu.sync_copy(src_h.at[t, pl.ds(c * _HC, _HC)], isrc)
        pltpu.sync_copy(dst_h.at[t, pl.ds(c * _HC, _HC)], idst)

        for si in range(S):
            g = gs[si]
            o = outs[si]

            _fill(buf0, _B, _FC, 0.0)
            for z in range(_RPT // _B):
                pltpu.sync_copy(buf0, acc.at[pl.ds(row0 + z * _B, _B)])
            plsc.subcore_barrier()

            pltpu.async_copy(g.at[isrc.at[0]], buf0, sem0)
            pltpu.async_copy(g.at[isrc.at[1]], buf1, sem1)

            def pipe(i, _, g=g):
                j0 = i * 2
                pltpu.make_async_copy(g.at[isrc.at[0]], buf0, sem0).wait()
                pltpu.sync_copy(buf0, acc.at[idst.at[j0]], add=True)

                @pl.when(i + 1 < _HP)
                def _():
                    pltpu.async_copy(g.at[isrc.at[j0 + 2]], buf0, sem0)

                pltpu.make_async_copy(g.at[isrc.at[1]], buf1, sem1).wait()
                pltpu.sync_copy(buf1, acc.at[idst.at[j0 + 1]], add=True)

                @pl.when(i + 1 < _HP)
                def _():
                    pltpu.async_copy(g.at[isrc.at[j0 + 3]], buf1, sem1)
                return 0

            lax.fori_loop(0, _HP, pipe, 0)
            plsc.subcore_barrier()
            for z in range(_RPT // _B):
                pltpu.sync_copy(acc.at[pl.ds(row0 + z * _B, _B)], buf0)
                pltpu.sync_copy(buf0, o.at[c, pl.ds(row0 + z * _B, _B)])

    return prop


_prop2 = _make_prop(2)


def _prop(src3, dst3, gs):
    outs = []
    for i in range(0, len(gs), 2):
        outs += _prop2(src3, dst3, gs[i], gs[i + 1])
    return outs


@functools.partial(
    pl.kernel,
    out_type=jax.ShapeDtypeStruct((_NC, NP, _FC), jnp.float32),
    mesh=_mesh,
    scratch_types=[
        pltpu.VMEM_SHARED((NP, _FC), jnp.float32),
        pltpu.VMEM((_HC, _B), jnp.int32),
        pltpu.VMEM((_B, _FC), jnp.float32),
    ],
    name="sc_deg")
def _degk(dst_h, out, acc, idst, onesb):
    c = lax.axis_index("c")
    t = lax.axis_index("s")
    row0 = t * _RPT
    pltpu.sync_copy(dst_h.at[t, pl.ds(c * _HC, _HC)], idst)
    _fill(onesb, _B, _FC, 0.0)
    for z in range(_RPT // _B):
        pltpu.sync_copy(onesb, acc.at[pl.ds(row0 + z * _B, _B)])
    _fill(onesb, _B, _FC, 1.0)
    plsc.subcore_barrier()

    def chunk(j, _):
        pltpu.sync_copy(onesb, acc.at[idst.at[j]], add=True)
        return 0

    lax.fori_loop(0, _HC, chunk, 0)
    plsc.subcore_barrier()
    for z in range(_RPT // _B):
        pltpu.sync_copy(acc.at[pl.ds(row0 + z * _B, _B)], onesb)
        pltpu.sync_copy(onesb, out.at[c, pl.ds(row0 + z * _B, _B)])


_BM = 2048


def _tc(body, n_out_cols, arrays):
    grid = (NP // _BM,)

    def spec(a):
        if a.ndim == 2 and a.shape[0] == NP:
            return pl.BlockSpec((_BM, a.shape[1]), lambda i: (i, 0))
        return pl.BlockSpec(a.shape, lambda i: (0,) * a.ndim)

    out_shape = [jax.ShapeDtypeStruct((NP, ncol), jnp.float32)
                 for ncol in n_out_cols]
    out_specs = [pl.BlockSpec((_BM, ncol), lambda i: (i, 0))
                 for ncol in n_out_cols]
    res = pl.pallas_call(
        body,
        grid=grid,
        in_specs=[spec(a) for a in arrays],
        out_specs=out_specs,
        out_shape=out_shape,
    )(*arrays)
    return res


def _dot(a, b):
    return jnp.dot(a, b, preferred_element_type=jnp.float32)


def _cat_combine(dv, b, sa, sb, gvals):
    parts = [dv * (x + y + g) for x, y, g in zip(sa, sb, gvals)]
    return jnp.maximum(jnp.concatenate(parts, axis=1) + b, 0.0)


def _split_out(g, outs):
    for i, o in enumerate(outs):
        o[...] = g[:, _FC * i:_FC * (i + 1)]


def _dinv_body(d0_ref, d1_ref, o_ref):
    deg = d0_ref[...][:, :1] + d1_ref[...][:, :1]
    o_ref[...] = lax.rsqrt(jnp.maximum(deg + 1.0, 1.0))


def _mm_scale_body(S):
    def body(h_ref, w_ref, dv_ref, *outs):
        g = _dot(h_ref[...], w_ref[...]) * dv_ref[...]
        _split_out(g, outs)
    return body


def _unpack_comb(refs, S_in):
    dv = refs[0][...]
    b = refs[1][...]
    sa = [r[...] for r in refs[2:2 + S_in]]
    sb = [r[...] for r in refs[2 + S_in:2 + 2 * S_in]]
    gvals = [r[...] for r in refs[2 + 2 * S_in:2 + 3 * S_in]]
    return dv, b, sa, sb, gvals, 2 + 3 * S_in


def _comb_mm_scale_body(S_in):
    def body(*refs):
        dv, b, sa, sb, gvals, k = _unpack_comb(refs, S_in)
        w = refs[k][...]
        outs = refs[k + 1:]
        h = _cat_combine(dv, b, sa, sb, gvals)
        _split_out(_dot(h, w) * dv, outs)
    return body


def _comb_pool_mm_scale_body(S_in, want_h):
    def body(*refs):
        dv, b, sa, sb, gvals, k = _unpack_comb(refs, S_in)
        wp, bp, wn = (r[...] for r in refs[k:k + 3])
        k += 3
        h = _cat_combine(dv, b, sa, sb, gvals)
        if want_h:
            refs[k][...] = h
            k += 1
        outs = refs[k:]
        hp = jnp.maximum(_dot(h, wp) + bp, 0.0)
        _split_out(_dot(hp, wn) * dv, outs)
    return body


def _comb_up_cat_mm_scale_body(S_in):
    def body(*refs):
        dv, b, sa, sb, gvals, k = _unpack_comb(refs, S_in)
        wu, bu, wa, wb, e1 = (r[...] for r in refs[k:k + 5])
        outs = refs[k + 5:]
        h = _cat_combine(dv, b, sa, sb, gvals)
        hu = jnp.maximum(_dot(h, wu) + bu, 0.0)
        g = (_dot(hu, wa) + _dot(e1, wb)) * dv
        _split_out(g, outs)
    return body


def _comb_out_body(S_in):
    def body(*refs):
        dv, b, sa, sb, gvals, k = _unpack_comb(refs, S_in)
        wo, bo = (r[...] for r in refs[k:k + 2])
        out = refs[k + 2]
        h = _cat_combine(dv, b, sa, sb, gvals)
        out[...] = jax.nn.sigmoid(_dot(h, wo) + bo)
    return body


def kernel(x, edge_index, params):
    p = params

    xp = jnp.zeros((NP, x.shape[1]), jnp.float32).at[:N].set(x)
    pad = _TILES * _CH * _B - E
    src3 = jnp.concatenate(
        [edge_index[0], jnp.full((pad,), NP - 1, edge_index.dtype)]
    ).reshape(_TILES, _CH, _B)
    dst3 = jnp.concatenate(
        [edge_index[1], jnp.full((pad,), NP - 1, edge_index.dtype)]
    ).reshape(_TILES, _CH, _B)

    def b2(name):
        return p[name]["b"].reshape(1, -1)

    def ab(s):
        return [u[0] for u in s], [u[1] for u in s]

    d2 = _degk(dst3)
    dv = _tc(_dinv_body, [1], [d2[0], d2[1]])[0]

    g1 = _tc(_mm_scale_body(4), [_FC] * 4, [xp, p["enc0_c1"]["W"], dv])
    a1, b1 = ab(_prop(src3, dst3, g1))
    g2 = _tc(_comb_mm_scale_body(4), [_FC] * 4,
             [dv, b2("enc0_c1"), *a1, *b1, *g1, p["enc0_c2"]["W"]])
    a2, b2_ = ab(_prop(src3, dst3, g2))
    g3 = _tc(_comb_pool_mm_scale_body(4, False), [_FC] * 4,
             [dv, b2("enc0_c2"), *a2, *b2_, *g2,
              p["pool0"]["W"], b2("pool0"), p["enc1_c1"]["W"]])

    a3, b3 = ab(_prop(src3, dst3, g3))
    g4 = _tc(_comb_mm_scale_body(4), [_FC] * 4,
             [dv, b2("enc1_c1"), *a3, *b3, *g3, p["enc1_c2"]["W"]])
    a4, b4 = ab(_prop(src3, dst3, g4))
    r5 = _tc(_comb_pool_mm_scale_body(4, True), [512] + [_FC] * 4,
             [dv, b2("enc1_c2"), *a4, *b4, *g4,
              p["pool1"]["W"], b2("pool1"), p["bot_c1"]["W"]])
    e1, g5 = r5[0], r5[1:]

    a5, b5 = ab(_prop(src3, dst3, list(g5)))
    g6 = _tc(_comb_mm_scale_body(4), [_FC] * 4,
             [dv, b2("bot_c1"), *a5, *b5, *g5, p["bot_c2"]["W"]])
    a6, b6 = ab(_prop(src3, dst3, g6))

    wd = p["dec0_c1"]["W"]
    wa, wb = wd[:256], wd[256:]
    g7 = _tc(_comb_up_cat_mm_scale_body(4), [_FC] * 2,
             [dv, b2("bot_c2"), *a6, *b6, *g6,
              p["up0"]["W"], b2("up0"), wa, wb, e1])
    a7, b7 = ab(_prop(src3, dst3, g7))
    g8 = _tc(_comb_mm_scale_body(2), [_FC] * 2,
             [dv, b2("dec0_c1"), *a7, *b7, *g7, p["dec0_c2"]["W"]])
    a8, b8 = ab(_prop(src3, dst3, g8))
    out = _tc(_comb_out_body(2), [256],
              [dv, b2("dec0_c2"), *a8, *b8, *g8, p["out"]["W"], b2("out")])[0]
    return out[:N]
